# baseline (speedup 1.0000x reference)
import functools

import jax
import jax.numpy as jnp
from jax import lax
from jax.experimental import pallas as pl
from jax.experimental.pallas import tpu as pltpu
from jax.experimental.pallas import tpu_sc as plsc

H = 256
W = 256
NPIX = H * W
FLOW_SCALING = 256.0
NCORES = 2
NSUB = 16
LANES = 16
CH = 1024
NCH = 32
CPT = CH * NCH
NPAD = NSUB * CPT
NB = 2
ACC = 4 * NPIX
ZCH = 4096
NROW = 4 * CH // 128


def _sc_body(ts_h, x_h, y_h, p_h, fx_h, fy_h, tref_h, out_h,
             tsb, xb, yb, pb, fxb, fyb,
             idxb, wcb, wtb, idxb2, wcb2, wtb2, zb,
             c0b, c1b, t0b, t1b,
             lossb, pallb, outb, trefb,
             acc, acc2, partials, sem, sem2):
  c = lax.axis_index("c")
  s = lax.axis_index("s")

  pltpu.sync_copy(tref_h.at[pl.ds(c * LANES, LANES)], trefb)
  trefv = trefb[...]

  def zstep(i, carry):
    zb[pl.ds(i * LANES, LANES)] = jnp.zeros((LANES,), jnp.float32)
    return carry
  lax.fori_loop(0, ZCH // LANES, zstep, 0)
  for k in range(ACC // NSUB // ZCH):
    off = s * (ACC // NSUB) + k * ZCH
    pltpu.sync_copy(zb, acc.at[pl.ds(off, ZCH)])
    pltpu.sync_copy(zb, acc2.at[pl.ds(off, ZCH)])
  plsc.subcore_barrier()

  bufs = (tsb, xb, yb, pb, fxb, fyb)
  srcs = (ts_h, x_h, y_h, p_h, fx_h, fy_h)
  stagings = ((idxb, wcb, wtb), (idxb2, wcb2, wtb2))

  def compute_chunk(b, slot, stag):
    sidx, swc, swt = stag

    def ostep(i8, icarry):
      i8h = i8 // 2
      i8m = i8 - i8h * 2
      for j in range(4):
        o = i8 * 64 + j * LANES
        col = i8m * 64 + j * LANES
        tsv = tsb[slot, pl.ds(o, LANES)]
        xv = xb[slot, pl.ds(o, LANES)]
        yv = yb[slot, pl.ds(o, LANES)]
        pv = pb[slot, pl.ds(o, LANES)]
        fxv = fxb[slot, pl.ds(o, LANES)]
        fyv = fyb[slot, pl.ds(o, LANES)]
        dtv = trefv - tsv
        wx = xv + dtv * fxv * FLOW_SCALING
        wy = yv + dtv * fyv * FLOW_SCALING
        xt = wx.astype(jnp.int32)
        x0 = jnp.where(xt.astype(jnp.float32) > wx, xt - 1, xt)
        yt = wy.astype(jnp.int32)
        y0 = jnp.where(yt.astype(jnp.float32) > wy, yt - 1, yt)
        dx = wx - x0.astype(jnp.float32)
        dy = wy - y0.astype(jnp.float32)
        omx = 1.0 - dx
        omy = 1.0 - dy
        polb = jnp.where(pv < 0.0, jnp.int32(1), jnp.int32(0))
        ib = (b * 2) * NPIX + polb * NPIX
        corners = ((0, 0, omx * omy), (1, 0, dx * omy),
                   (0, 1, omx * dy), (1, 1, dx * dy))
        for k, (kx, ky, wgt) in enumerate(corners):
          xc = x0 + kx
          yc = y0 + ky
          valid = ((xc >= 0) & (xc <= W - 1)
                   & (yc >= 0) & (yc <= H - 1))
          xi = jnp.minimum(jnp.maximum(xc, 0), W - 1)
          yi = jnp.minimum(jnp.maximum(yc, 0), H - 1)
          lin = ib + yi * W + xi
          wv = jnp.where(valid, wgt, jnp.float32(0))
          r = 8 * k + i8h
          sidx[r, pl.ds(col, LANES)] = lin
          swc[r, pl.ds(col, LANES)] = wv
          swt[r, pl.ds(col, LANES)] = wv * tsv
      return icarry
    lax.fori_loop(0, CH // 64, ostep, 0)

  def fire_streams(stag):
    sidx, swc, swt = stag
    descs = []
    for j in range(NROW):
      descs.append(pltpu.async_copy(
          swc.at[j], acc.at[sidx.at[j]], sem, add=True))
      descs.append(pltpu.async_copy(
          swt.at[j], acc2.at[sidx.at[j]], sem, add=True))
    return descs

  for b in range(NB):
    hbase = b * NPAD + s * CPT

    for sl in range(2):
      for bf, sh in zip(bufs, srcs):
        pltpu.sync_copy(sh.at[pl.ds(hbase + sl * CH, CH)], bf.at[sl])

    def cstep(g, carry):
      gp = g - (g // 2) * 2
      sl0 = 2 * gp
      np0 = 2 - sl0
      cn = jnp.where(2 * g + 2 > NCH - 2, NCH - 2, 2 * g + 2)
      idescs = []
      for bf, sh in zip(bufs, srcs):
        idescs.append(pltpu.async_copy(
            sh.at[pl.ds(hbase + cn * CH, CH)], bf.at[np0], sem2))
        idescs.append(pltpu.async_copy(
            sh.at[pl.ds(hbase + (cn + 1) * CH, CH)], bf.at[np0 + 1], sem2))

      compute_chunk(b, sl0, stagings[0])
      descs0 = fire_streams(stagings[0])
      compute_chunk(b, sl0 + 1, stagings[1])
      descs1 = fire_streams(stagings[1])
      for d in descs0 + descs1 + idescs:
        d.wait()
      return carry
    lax.fori_loop(0, NCH // 2, cstep, 0)

  plsc.subcore_barrier()

  pix0 = s * ZCH
  for b in range(NB):
    pltpu.sync_copy(acc.at[pl.ds((b * 2 + 0) * NPIX + pix0, ZCH)], c0b)
    pltpu.sync_copy(acc.at[pl.ds((b * 2 + 1) * NPIX + pix0, ZCH)], c1b)
    pltpu.sync_copy(acc2.at[pl.ds((b * 2 + 0) * NPIX + pix0, ZCH)], t0b)
    pltpu.sync_copy(acc2.at[pl.ds((b * 2 + 1) * NPIX + pix0, ZCH)], t1b)

    def lstep(i, carry):
      al, an = carry
      o = i * LANES
      c0 = c0b[pl.ds(o, LANES)]
      c1 = c1b[pl.ds(o, LANES)]
      t0 = t0b[pl.ds(o, LANES)]
      t1 = t1b[pl.ds(o, LANES)]
      r0 = t0 / (c0 + 1e-9)
      r1 = t1 / (c1 + 1e-9)
      al = al + r0 * r0 + r1 * r1
      an = an + jnp.where((c0 + c1) > 0.0, 1.0, 0.0).astype(jnp.float32)
      return al, an
    zero16 = jnp.zeros((LANES,), jnp.float32)
    al, an = lax.fori_loop(0, ZCH // LANES, lstep, (zero16, zero16))
    lossb[pl.ds(b * LANES, LANES)] = al
    lossb[pl.ds((2 + b) * LANES, LANES)] = an

  pltpu.sync_copy(lossb, partials.at[pl.ds(s * 64, 64)])
  plsc.subcore_barrier()

  @pl.when(s == 0)
  def _():
    pltpu.sync_copy(partials, pallb)
    for rw in range(4):
      acc = jnp.zeros((LANES,), jnp.float32)
      for t in range(NSUB):
        acc = acc + pallb[pl.ds(t * 64 + rw * LANES, LANES)]
      outb[pl.ds(rw * LANES, LANES)] = acc
    pltpu.sync_copy(outb, out_h.at[pl.ds(c * 64, 64)])


@functools.cache
def _build_sc_kernel():
  return pl.kernel(
    _sc_body,
    out_type=jax.ShapeDtypeStruct((NCORES * 64,), jnp.float32),
    mesh=plsc.VectorSubcoreMesh(core_axis_name="c", subcore_axis_name="s",
                                num_cores=NCORES, num_subcores=NSUB),
    scratch_types=[
        pltpu.VMEM((4, CH), jnp.float32),
        pltpu.VMEM((4, CH), jnp.float32),
        pltpu.VMEM((4, CH), jnp.float32),
        pltpu.VMEM((4, CH), jnp.float32),
        pltpu.VMEM((4, CH), jnp.float32),
        pltpu.VMEM((4, CH), jnp.float32),
        pltpu.VMEM((NROW, 128), jnp.int32),
        pltpu.VMEM((NROW, 128), jnp.float32),
        pltpu.VMEM((NROW, 128), jnp.float32),
        pltpu.VMEM((NROW, 128), jnp.int32),
        pltpu.VMEM((NROW, 128), jnp.float32),
        pltpu.VMEM((NROW, 128), jnp.float32),
        pltpu.VMEM((ZCH,), jnp.float32),
        pltpu.VMEM((ZCH,), jnp.float32),
        pltpu.VMEM((ZCH,), jnp.float32),
        pltpu.VMEM((ZCH,), jnp.float32),
        pltpu.VMEM((ZCH,), jnp.float32),
        pltpu.VMEM((64,), jnp.float32),
        pltpu.VMEM((NSUB * 64,), jnp.float32),
        pltpu.VMEM((64,), jnp.float32),
        pltpu.VMEM((LANES,), jnp.float32),
        pltpu.VMEM_SHARED((ACC,), jnp.float32),
        pltpu.VMEM_SHARED((ACC,), jnp.float32),
        pltpu.VMEM_SHARED((NSUB * 64,), jnp.float32),
        pltpu.SemaphoreType.DMA,
        pltpu.SemaphoreType.DMA,
    ],
  )


def _pad_flat(a, fill):
  b, n = a.shape
  pad = jnp.full((b, NPAD - n), fill, jnp.float32)
  return jnp.concatenate([a, pad], axis=1).reshape(-1)


def kernel(event_list, flow_list, max_ts):
  mt = jnp.asarray(max_ts, jnp.float32)
  ts = _pad_flat(event_list[..., 0], 0.0)
  x = _pad_flat(event_list[..., 1], -1.0e6)
  y = _pad_flat(event_list[..., 2], 0.0)
  p = _pad_flat(event_list[..., 3], 0.0)
  fx = _pad_flat(flow_list[..., 0], 0.0)
  fy = _pad_flat(flow_list[..., 1], 0.0)
  tref = jnp.concatenate([jnp.full((LANES,), mt, jnp.float32),
                          jnp.zeros((LANES,), jnp.float32)])

  out = _build_sc_kernel()(ts, x, y, p, fx, fy, tref)
  sums = out.reshape(NCORES, 4, LANES).sum(-1)
  loss = sums[:, 0:2]
  nnz = sums[:, 2:4]
  return (loss / nnz).sum() / (mt ** 2)


# speedup vs baseline: 1.4403x; 1.4403x over previous
import functools

import jax
import jax.numpy as jnp
from jax import lax
from jax.experimental import pallas as pl
from jax.experimental.pallas import tpu as pltpu
from jax.experimental.pallas import tpu_sc as plsc

H = 256
W = 256
NPIX = H * W
FLOW_SCALING = 256.0
NCORES = 2
NSUB = 16
LANES = 16
CH = 1024
NCH = 31
CPT = CH * NCH
NPAD = NSUB * CPT
NB = 2
ACC = 4 * NPIX
ZCH = 4096
NROW = 4 * CH // 128


def _sc_body(ts_h, x_h, y_h, p_h, fx_h, fy_h, tref_h, out_h,
             tsb, xb, yb, pb, fxb, fyb,
             idxb, wcb, wtb, zb,
             c0b, c1b, t0b, t1b,
             lossb, pallb, outb, trefb,
             acc, acc2, partials, sem, sem2):
  c = lax.axis_index("c")
  s = lax.axis_index("s")

  pltpu.sync_copy(tref_h.at[pl.ds(c * LANES, LANES)], trefb)
  trefv = trefb[...]

  def zstep(i, carry):
    zb[pl.ds(i * LANES, LANES)] = jnp.zeros((LANES,), jnp.float32)
    return carry
  lax.fori_loop(0, ZCH // LANES, zstep, 0)
  for k in range(ACC // NSUB // ZCH):
    off = s * (ACC // NSUB) + k * ZCH
    pltpu.sync_copy(zb, acc.at[pl.ds(off, ZCH)])
    pltpu.sync_copy(zb, acc2.at[pl.ds(off, ZCH)])
  plsc.subcore_barrier()

  bufs = (tsb, xb, yb, pb, fxb, fyb)
  srcs = (ts_h, x_h, y_h, p_h, fx_h, fy_h)
  stagings = ((idxb, wcb, wtb),)

  def compute_chunk(b, slot, stag):
    sidx, swc, swt = stag

    def ostep(i8, icarry):
      for j in range(8):
        o = i8 * 128 + j * LANES
        col = j * LANES
        tsv = tsb[slot, pl.ds(o, LANES)]
        xv = xb[slot, pl.ds(o, LANES)]
        yv = yb[slot, pl.ds(o, LANES)]
        pv = pb[slot, pl.ds(o, LANES)]
        fxv = fxb[slot, pl.ds(o, LANES)]
        fyv = fyb[slot, pl.ds(o, LANES)]
        dtv = trefv - tsv
        wx = xv + dtv * fxv
        wy = yv + dtv * fyv
        xt = wx.astype(jnp.int32)
        x0 = jnp.where(xt.astype(jnp.float32) > wx, xt - 1, xt)
        yt = wy.astype(jnp.int32)
        y0 = jnp.where(yt.astype(jnp.float32) > wy, yt - 1, yt)
        dx = wx - x0.astype(jnp.float32)
        dy = wy - y0.astype(jnp.float32)
        omx = 1.0 - dx
        omy = 1.0 - dy
        polb = jnp.where(pv < 0.0, jnp.int32(1), jnp.int32(0))
        ib = (b * 2) * NPIX + polb * NPIX
        x1 = x0 + 1
        y1 = y0 + 1
        vx0 = (x0 >= 0) & (x0 <= W - 1)
        vx1 = (x1 >= 0) & (x1 <= W - 1)
        vy0 = (y0 >= 0) & (y0 <= H - 1)
        vy1 = (y1 >= 0) & (y1 <= H - 1)
        xi0 = jnp.minimum(jnp.maximum(x0, 0), W - 1)
        xi1 = jnp.minimum(jnp.maximum(x1, 0), W - 1)
        row0 = ib + jnp.minimum(jnp.maximum(y0, 0), H - 1) * W
        row1 = ib + jnp.minimum(jnp.maximum(y1, 0), H - 1) * W
        corners = ((xi0, row0, vx0 & vy0, omx * omy),
                   (xi1, row0, vx1 & vy0, dx * omy),
                   (xi0, row1, vx0 & vy1, omx * dy),
                   (xi1, row1, vx1 & vy1, dx * dy))
        for k, (xi, row, valid, wgt) in enumerate(corners):
          lin = row + xi
          wv = jnp.where(valid, wgt, jnp.float32(0))
          r = 8 * k + i8
          sidx[r, pl.ds(col, LANES)] = lin
          swc[r, pl.ds(col, LANES)] = wv
          swt[r, pl.ds(col, LANES)] = wv * tsv
      return icarry
    lax.fori_loop(0, CH // 128, ostep, 0)

  def fire_streams(stag):
    sidx, swc, swt = stag
    descs = []
    for j in range(NROW):
      descs.append(pltpu.async_copy(
          swc.at[j], acc.at[sidx.at[j]], sem, add=True))
      descs.append(pltpu.async_copy(
          swt.at[j], acc2.at[sidx.at[j]], sem, add=True))
    return descs

  for b in range(NB):
    hbase = b * NPAD + s * CPT

    for bf, sh in zip(bufs, srcs):
      pltpu.sync_copy(sh.at[pl.ds(hbase, CH)], bf.at[0])

    def cstep(ci, carry):
      p = ci - (ci // 2) * 2
      pn = 1 - p
      cnext = jnp.where(ci + 1 > NCH - 1, NCH - 1, ci + 1)
      nbase = hbase + cnext * CH
      idescs = [pltpu.async_copy(sh.at[pl.ds(nbase, CH)], bf.at[pn], sem2)
                for bf, sh in zip(bufs, srcs)]

      compute_chunk(b, p, stagings[0])
      descs = fire_streams(stagings[0])
      for d in descs:
        d.wait()
      for d in idescs:
        d.wait()
      return carry
    lax.fori_loop(0, NCH, cstep, 0)

  plsc.subcore_barrier()

  pix0 = s * ZCH
  for b in range(NB):
    pltpu.sync_copy(acc.at[pl.ds((b * 2 + 0) * NPIX + pix0, ZCH)], c0b)
    pltpu.sync_copy(acc.at[pl.ds((b * 2 + 1) * NPIX + pix0, ZCH)], c1b)
    pltpu.sync_copy(acc2.at[pl.ds((b * 2 + 0) * NPIX + pix0, ZCH)], t0b)
    pltpu.sync_copy(acc2.at[pl.ds((b * 2 + 1) * NPIX + pix0, ZCH)], t1b)

    def lstep(i, carry):
      al, an = carry
      o = i * LANES
      c0 = c0b[pl.ds(o, LANES)]
      c1 = c1b[pl.ds(o, LANES)]
      t0 = t0b[pl.ds(o, LANES)]
      t1 = t1b[pl.ds(o, LANES)]
      r0 = t0 / (c0 + 1e-9)
      r1 = t1 / (c1 + 1e-9)
      al = al + r0 * r0 + r1 * r1
      an = an + jnp.where((c0 + c1) > 0.0, 1.0, 0.0).astype(jnp.float32)
      return al, an
    zero16 = jnp.zeros((LANES,), jnp.float32)
    al, an = lax.fori_loop(0, ZCH // LANES, lstep, (zero16, zero16))
    lossb[pl.ds(b * LANES, LANES)] = al
    lossb[pl.ds((2 + b) * LANES, LANES)] = an

  pltpu.sync_copy(lossb, partials.at[pl.ds(s * 64, 64)])
  plsc.subcore_barrier()

  @pl.when(s == 0)
  def _():
    pltpu.sync_copy(partials, pallb)
    for rw in range(4):
      acc = jnp.zeros((LANES,), jnp.float32)
      for t in range(NSUB):
        acc = acc + pallb[pl.ds(t * 64 + rw * LANES, LANES)]
      outb[pl.ds(rw * LANES, LANES)] = acc
    pltpu.sync_copy(outb, out_h.at[pl.ds(c * 64, 64)])


@functools.cache
def _build_sc_kernel():
  return pl.kernel(
    _sc_body,
    out_type=jax.ShapeDtypeStruct((NCORES * 64,), jnp.float32),
    mesh=plsc.VectorSubcoreMesh(core_axis_name="c", subcore_axis_name="s",
                                num_cores=NCORES, num_subcores=NSUB),
    scratch_types=[
        pltpu.VMEM((4, CH), jnp.float32),
        pltpu.VMEM((4, CH), jnp.float32),
        pltpu.VMEM((4, CH), jnp.float32),
        pltpu.VMEM((4, CH), jnp.float32),
        pltpu.VMEM((4, CH), jnp.float32),
        pltpu.VMEM((4, CH), jnp.float32),
        pltpu.VMEM((NROW, 128), jnp.int32),
        pltpu.VMEM((NROW, 128), jnp.float32),
        pltpu.VMEM((NROW, 128), jnp.float32),
        pltpu.VMEM((ZCH,), jnp.float32),
        pltpu.VMEM((ZCH,), jnp.float32),
        pltpu.VMEM((ZCH,), jnp.float32),
        pltpu.VMEM((ZCH,), jnp.float32),
        pltpu.VMEM((ZCH,), jnp.float32),
        pltpu.VMEM((64,), jnp.float32),
        pltpu.VMEM((NSUB * 64,), jnp.float32),
        pltpu.VMEM((64,), jnp.float32),
        pltpu.VMEM((LANES,), jnp.float32),
        pltpu.VMEM_SHARED((ACC,), jnp.float32),
        pltpu.VMEM_SHARED((ACC,), jnp.float32),
        pltpu.VMEM_SHARED((NSUB * 64,), jnp.float32),
        pltpu.SemaphoreType.DMA,
        pltpu.SemaphoreType.DMA,
    ],
  )


def _pad_flat(a, fill):
  b, n = a.shape
  pad = jnp.full((b, NPAD - n), fill, jnp.float32)
  return jnp.concatenate([a, pad], axis=1).reshape(-1)


def kernel(event_list, flow_list, max_ts):
  mt = jnp.asarray(max_ts, jnp.float32)
  ts = _pad_flat(event_list[..., 0], 0.0)
  x = _pad_flat(event_list[..., 1], -1.0e6)
  y = _pad_flat(event_list[..., 2], 0.0)
  p = _pad_flat(event_list[..., 3], 0.0)
  fx = _pad_flat(flow_list[..., 0] * FLOW_SCALING, 0.0)
  fy = _pad_flat(flow_list[..., 1] * FLOW_SCALING, 0.0)
  tref = jnp.concatenate([jnp.full((LANES,), mt, jnp.float32),
                          jnp.zeros((LANES,), jnp.float32)])

  out = _build_sc_kernel()(ts, x, y, p, fx, fy, tref)
  sums = out.reshape(NCORES, 4, LANES).sum(-1)
  loss = sums[:, 0:2]
  nnz = sums[:, 2:4]
  return (loss / nnz).sum() / (mt ** 2)
